# baseline (speedup 1.0000x reference)
import functools

import jax
import jax.numpy as jnp
from jax import lax
from jax.experimental import pallas as pl
from jax.experimental.pallas import tpu as pltpu

GJ = 8


def _gather_block_body(idx_s, *refs):
    ins = refs[:-1]
    out_ref = refs[-1]
    g = pl.program_id(0)
    np_ = len(ins)
    for j, in_ref in enumerate(ins):
        m = idx_s[np_ * g + j] % GJ
        out_ref[:, pl.ds(j, 1), :] = in_ref[:, pl.ds(m, 1), :]


def _band_spec(j, P, B, C):
    return pl.BlockSpec(
        (C, GJ, B),
        functools.partial(
            lambda jj, g, b, idx: (0, idx[P * g + jj] // GJ, b), j),
    )


def kernel(vertices, vert_idx):
    B, V, C = vertices.shape
    (K,) = vert_idx.shape
    P = 48
    KP = ((K + P - 1) // P) * P
    idxp = jnp.concatenate(
        [vert_idx.astype(jnp.int32), jnp.zeros((KP - K,), jnp.int32)])
    vt = jnp.transpose(vertices, (2, 1, 0))
    out_t = pl.pallas_call(
        _gather_block_body,
        grid_spec=pltpu.PrefetchScalarGridSpec(
            num_scalar_prefetch=1,
            grid=(KP // P, 1),
            in_specs=[_band_spec(j, P, B, C) for j in range(P)],
            out_specs=pl.BlockSpec((C, P, B),
                                   lambda g, b, idx: (0, g, b)),
        ),
        out_shape=jax.ShapeDtypeStruct((C, K, B), jnp.float32),
    )(idxp, *([vt] * P))
    return jnp.transpose(out_t, (2, 1, 0))


# speedup vs baseline: 1.1079x; 1.1079x over previous
import functools

import jax
import jax.numpy as jnp
from jax import lax
from jax.experimental import pallas as pl
from jax.experimental.pallas import tpu as pltpu

GJ = 8


def _gather_block_body(idx_s, *refs):
    ins = refs[:-1]
    out_ref = refs[-1]
    g = pl.program_id(0)
    np_ = len(ins)
    for j, in_ref in enumerate(ins):
        m = idx_s[np_ * g + j] % GJ
        out_ref[:, pl.ds(j, 1), :] = in_ref[:, pl.ds(m, 1), :]


def _band_spec(j, P, B, C):
    return pl.BlockSpec(
        (C, GJ, B),
        functools.partial(
            lambda jj, g, b, idx: (0, idx[P * g + jj] // GJ, b), j),
    )


def kernel(vertices, vert_idx):
    B, V, C = vertices.shape
    (K,) = vert_idx.shape
    P = 16
    KP = ((K + P - 1) // P) * P
    idxp = jnp.concatenate(
        [vert_idx.astype(jnp.int32), jnp.zeros((KP - K,), jnp.int32)])
    vt = jnp.transpose(vertices, (2, 1, 0))
    out_t = pl.pallas_call(
        _gather_block_body,
        grid_spec=pltpu.PrefetchScalarGridSpec(
            num_scalar_prefetch=1,
            grid=(KP // P, 1),
            in_specs=[_band_spec(j, P, B, C) for j in range(P)],
            out_specs=pl.BlockSpec((C, P, B),
                                   lambda g, b, idx: (0, g, b)),
        ),
        out_shape=jax.ShapeDtypeStruct((C, K, B), jnp.float32),
    )(idxp, *([vt] * P))
    return jnp.transpose(out_t, (2, 1, 0))
